# Initial kernel scaffold; baseline (speedup 1.0000x reference)
import jax
import jax.numpy as jnp
from jax.experimental import pallas as pl


def kernel(pred, target, batch):
    raise NotImplementedError("write your pallas kernel here")


# speedup vs baseline: 13.1819x; 13.1819x over previous
import jax
import jax.numpy as jnp
from jax import lax
from jax.experimental import pallas as pl
from jax.experimental.pallas import tpu as pltpu
from jax.experimental.pallas import tpu_sc as plsc

N = 100000
SEG = 512
LANES = 16
NC, NS = 2, 16
NW = NC * NS
NV_TOTAL = N // LANES
NV_LO = NV_TOTAL // NW
EXTRA = NV_TOTAL - NV_LO * NW
CHUNK_LO = NV_LO * LANES
CHUNK_HI = (NV_LO + 1) * LANES
ACC_W = 2 * SEG


def _stage1_body(pred_hbm, target_hbm, batch_hbm, out_hbm,
                 pred_v, target_v, batch_v, acc, partial_v):
    wid = lax.axis_index("c") * NS + lax.axis_index("s")
    has_extra = wid < EXTRA
    base = wid * CHUNK_LO + jnp.minimum(wid, EXTRA) * LANES
    nv = NV_LO + has_extra.astype(jnp.int32)

    @pl.when(has_extra)
    def _():
        pltpu.sync_copy(pred_hbm.at[pl.ds(base, CHUNK_HI)], pred_v)
        pltpu.sync_copy(target_hbm.at[pl.ds(base, CHUNK_HI)], target_v)
        pltpu.sync_copy(batch_hbm.at[pl.ds(base, CHUNK_HI)], batch_v)

    @pl.when(jnp.logical_not(has_extra))
    def _():
        pltpu.sync_copy(pred_hbm.at[pl.ds(base, CHUNK_LO)],
                        pred_v.at[pl.ds(0, CHUNK_LO)])
        pltpu.sync_copy(target_hbm.at[pl.ds(base, CHUNK_LO)],
                        target_v.at[pl.ds(0, CHUNK_LO)])
        pltpu.sync_copy(batch_hbm.at[pl.ds(base, CHUNK_LO)],
                        batch_v.at[pl.ds(0, CHUNK_LO)])

    zero = jnp.zeros((LANES,), jnp.float32)

    def zero_body(cb, carry):
        off = cb * LANES
        for r in range(LANES):
            acc[r, pl.ds(off, LANES)] = zero
        return carry

    lax.fori_loop(0, ACC_W // LANES, zero_body, 0)

    row = lax.iota(jnp.int32, LANES)

    def body(j, carry):
        off = j * LANES
        p = pred_v[pl.ds(off, LANES)]
        t = target_v[pl.ds(off, LANES)]
        b = batch_v[pl.ds(off, LANES)]
        plsc.addupdate_scatter(acc, [row, b], p * t)
        plsc.addupdate_scatter(acc, [row, b + SEG], p + t)
        return carry

    lax.fori_loop(0, nv, body, 0)

    def red_body(cb, carry):
        off = cb * LANES
        v = acc[0, pl.ds(off, LANES)]
        for r in range(1, LANES):
            v = v + acc[r, pl.ds(off, LANES)]
        partial_v[pl.ds(off, LANES)] = v
        return carry

    lax.fori_loop(0, ACC_W // LANES, red_body, 0)
    pltpu.sync_copy(partial_v, out_hbm.at[wid])


_stage1 = pl.kernel(
    _stage1_body,
    out_type=jax.ShapeDtypeStruct((NW, ACC_W), jnp.float32),
    mesh=plsc.VectorSubcoreMesh(core_axis_name="c", subcore_axis_name="s",
                                num_cores=NC, num_subcores=NS),
    scratch_types=[
        pltpu.VMEM((CHUNK_HI,), jnp.float32),
        pltpu.VMEM((CHUNK_HI,), jnp.float32),
        pltpu.VMEM((CHUNK_HI,), jnp.int32),
        pltpu.VMEM((LANES, ACC_W), jnp.float32),
        pltpu.VMEM((ACC_W,), jnp.float32),
    ],
    compiler_params=pltpu.CompilerParams(needs_layout_passes=False),
)


def _stage2_body(p_ref, o_ref):
    x = p_ref[...]
    inter = jnp.sum(x[:, :SEG], axis=0, keepdims=True)
    denom = jnp.sum(x[:, SEG:], axis=0, keepdims=True)
    dice = (2.0 * inter + 1.0) / (denom + 1.0)
    o_ref[0, 0] = jnp.sum(1.0 - dice)


_stage2 = pl.pallas_call(
    _stage2_body,
    out_shape=jax.ShapeDtypeStruct((1, 1), jnp.float32),
    out_specs=pl.BlockSpec(memory_space=pltpu.SMEM),
)


def kernel(pred, target, batch):
    partials = _stage1(pred, target, batch.astype(jnp.int32))
    return _stage2(partials)[0, 0]
